# Initial kernel scaffold; baseline (speedup 1.0000x reference)
import jax
import jax.numpy as jnp
from jax.experimental import pallas as pl


def kernel(preprocessed_tags, embedding):
    raise NotImplementedError("write your pallas kernel here")


# speedup vs baseline: 3.0070x; 3.0070x over previous
import functools

import jax
import jax.numpy as jnp
from jax import lax
from jax.experimental import pallas as pl
from jax.experimental.pallas import tpu as pltpu
from jax.experimental.pallas import tpu_sc as plsc

VOCAB = 50
D = 16
B = 16384
L = 200
N = B * L
RCOLS = 128
R = N // RCOLS
NC = 2
NS = 16
NW = NC * NS
RW = R // NW
CHUNK = 16
NCHUNK = RW // CHUNK


def _body(tags_ref, table_ref, out_ref, relu_ref, tab_v, idx_v, rows_v, sem):
    wid = lax.axis_index("s") * NC + lax.axis_index("c")

    pltpu.sync_copy(table_ref, tab_v)
    for i in range(VOCAB):
        tab_v[i, :] = jnp.maximum(tab_v[i, :], 0.0)
    pltpu.sync_copy(tab_v, relu_ref)

    @pl.loop(0, NCHUNK)
    def _chunk(c):
        rbase = wid * RW + c * CHUNK
        pltpu.sync_copy(tags_ref.at[pl.ds(rbase, CHUNK)], idx_v)
        cps = [
            pltpu.async_copy(relu_ref.at[idx_v.at[k]], rows_v.at[k], sem)
            for k in range(CHUNK)
        ]
        for cp in cps:
            cp.wait()
        pltpu.sync_copy(rows_v, out_ref.at[pl.ds(rbase, CHUNK)])


@jax.jit
def _run(tags2d, embedding):
    mesh = plsc.VectorSubcoreMesh(
        core_axis_name="c", subcore_axis_name="s", num_cores=NC, num_subcores=NS
    )
    kern = pl.kernel(
        _body,
        out_type=(
            jax.ShapeDtypeStruct((R, RCOLS, D), jnp.float32),
            jax.ShapeDtypeStruct((VOCAB, D), jnp.float32),
        ),
        mesh=mesh,
        scratch_types=[
            pltpu.VMEM((VOCAB, D), jnp.float32),
            pltpu.VMEM((CHUNK, RCOLS), jnp.int32),
            pltpu.VMEM((CHUNK, RCOLS, D), jnp.float32),
            pltpu.SemaphoreType.DMA,
        ],
        compiler_params=pltpu.CompilerParams(use_tc_tiling_on_sc=False),
    )
    out3, _ = kern(tags2d, embedding)
    return out3


def kernel(preprocessed_tags, embedding):
    tags2d = preprocessed_tags.reshape(R, RCOLS)
    out3 = _run(tags2d, embedding)
    return out3.reshape(B, L, D)


# speedup vs baseline: 3.0071x; 1.0000x over previous
import functools

import jax
import jax.numpy as jnp
from jax import lax
from jax.experimental import pallas as pl
from jax.experimental.pallas import tpu as pltpu
from jax.experimental.pallas import tpu_sc as plsc

VOCAB = 50
D = 16
B = 16384
L = 200
N = B * L
RCOLS = 512
R = N // RCOLS
NC = 2
NS = 16
NW = NC * NS
RW = R // NW
CHUNK = 4
NCHUNK = RW // CHUNK


def _body(tags_ref, table_ref, out_ref, relu_ref, tab_v, idx_v, rows_v, sem):
    wid = lax.axis_index("s") * NC + lax.axis_index("c")

    pltpu.sync_copy(table_ref, tab_v)
    for i in range(VOCAB):
        tab_v[i, :] = jnp.maximum(tab_v[i, :], 0.0)
    pltpu.sync_copy(tab_v, relu_ref)

    @pl.loop(0, NCHUNK)
    def _chunk(c):
        rbase = wid * RW + c * CHUNK
        pltpu.sync_copy(tags_ref.at[pl.ds(rbase, CHUNK)], idx_v)
        cps = [
            pltpu.async_copy(relu_ref.at[idx_v.at[k]], rows_v.at[k], sem)
            for k in range(CHUNK)
        ]
        for cp in cps:
            cp.wait()
        pltpu.sync_copy(rows_v, out_ref.at[pl.ds(rbase, CHUNK)])


@jax.jit
def _run(tags2d, embedding):
    mesh = plsc.VectorSubcoreMesh(
        core_axis_name="c", subcore_axis_name="s", num_cores=NC, num_subcores=NS
    )
    kern = pl.kernel(
        _body,
        out_type=(
            jax.ShapeDtypeStruct((R, RCOLS, D), jnp.float32),
            jax.ShapeDtypeStruct((VOCAB, D), jnp.float32),
        ),
        mesh=mesh,
        scratch_types=[
            pltpu.VMEM((VOCAB, D), jnp.float32),
            pltpu.VMEM((CHUNK, RCOLS), jnp.int32),
            pltpu.VMEM((CHUNK, RCOLS, D), jnp.float32),
            pltpu.SemaphoreType.DMA,
        ],
        compiler_params=pltpu.CompilerParams(use_tc_tiling_on_sc=False),
    )
    out3, _ = kern(tags2d, embedding)
    return out3


def kernel(preprocessed_tags, embedding):
    tags2d = preprocessed_tags.reshape(R, RCOLS)
    out3 = _run(tags2d, embedding)
    return out3.reshape(B, L, D)
